# Initial kernel scaffold; baseline (speedup 1.0000x reference)
import jax
import jax.numpy as jnp
from jax.experimental import pallas as pl


def kernel(h, pos, idx, W1, b1, W2, b2, gamma, beta):
    raise NotImplementedError("write your pallas kernel here")


# speedup vs baseline: 17.6844x; 17.6844x over previous
import functools

import jax
import jax.numpy as jnp
from jax import lax
from jax.experimental import pallas as pl
from jax.experimental.pallas import tpu as pltpu
from jax.experimental.pallas import tpu_sc as plsc


_SQRT_HALF = 0.7071067811865476


def _gelu(x):
    u = x * _SQRT_HALF
    s = jnp.sign(u)
    au = jnp.abs(u)
    t = 1.0 / (1.0 + 0.3275911 * au)
    poly = t * (0.254829592 + t * (-0.284496736 + t * (1.421413741
           + t * (-1.453152027 + t * 1.061405429))))
    erf = s * (1.0 - poly * jnp.exp(-au * au))
    return 0.5 * x * (1.0 + erf)


def _stage1_body(n_per_batch, blk, h_ref, p_ref, i_ref, w1_ref, b1_ref,
                 a_ref, g_ref, ig_ref):
    C = h_ref.shape[1]
    hb = h_ref[...]
    w1a = w1_ref[0:C, :]
    w1b = w1_ref[C:2 * C, :]
    r = (p_ref[:, 0:1] * w1_ref[2 * C:2 * C + 1, :]
         + p_ref[:, 1:2] * w1_ref[2 * C + 1:2 * C + 2, :]
         + p_ref[:, 2:3] * w1_ref[2 * C + 2:2 * C + 3, :])
    g_ref[...] = jnp.dot(hb, w1b, preferred_element_type=jnp.float32) + r
    a_ref[...] = (jnp.dot(hb, w1a - w1b, preferred_element_type=jnp.float32)
                  + b1_ref[...] - r)
    pid = pl.program_id(0)
    ig_ref[...] = i_ref[...] + (pid * blk) // n_per_batch * n_per_batch


def _stage3_body(nodes, k, h_out, gath_ref, a_ref, w2_ref, b2_ref,
                 gm_ref, bt_ref, o_ref):
    z = gath_ref[...].reshape(nodes, k, h_out) + a_ref[...][:, None, :]
    x1 = _gelu(z).reshape(nodes * k, h_out)
    y = jnp.dot(x1, w2_ref[...], preferred_element_type=jnp.float32) + b2_ref[...]
    x2 = _gelu(y).reshape(nodes, k, h_out)
    m = jnp.max(x2, axis=1)
    mu = jnp.mean(m, axis=-1, keepdims=True)
    var = jnp.mean((m - mu) ** 2, axis=-1, keepdims=True)
    o_ref[...] = (m - mu) * lax.rsqrt(var + 1e-5) * gm_ref[...] + bt_ref[...]


def kernel(h, pos, idx, W1, b1, W2, b2, gamma, beta):
    B, N, C = h.shape
    K = idx.shape[-1]
    H = W1.shape[1]
    OUT = W2.shape[1]
    BN = B * N
    E = BN * K

    hf = h.reshape(BN, C)
    pf = pos.reshape(BN, 3)
    idxf = idx.reshape(BN, K)

    BLK1 = 1000
    grid1 = BN // BLK1
    a, g, idxg = pl.pallas_call(
        functools.partial(_stage1_body, N, BLK1),
        grid=(grid1,),
        in_specs=[
            pl.BlockSpec((BLK1, C), lambda i: (i, 0)),
            pl.BlockSpec((BLK1, 3), lambda i: (i, 0)),
            pl.BlockSpec((BLK1, K), lambda i: (i, 0)),
            pl.BlockSpec((2 * C + 3, H), lambda i: (0, 0)),
            pl.BlockSpec((1, H), lambda i: (0, 0)),
        ],
        out_specs=[
            pl.BlockSpec((BLK1, H), lambda i: (i, 0)),
            pl.BlockSpec((BLK1, H), lambda i: (i, 0)),
            pl.BlockSpec((BLK1, K), lambda i: (i, 0)),
        ],
        out_shape=[
            jax.ShapeDtypeStruct((BN, H), jnp.float32),
            jax.ShapeDtypeStruct((BN, H), jnp.float32),
            jax.ShapeDtypeStruct((BN, K), jnp.int32),
        ],
    )(hf, pf, idxf, W1, b1.reshape(1, H))

    info = plsc.get_sparse_core_info()
    NC, NS = info.num_cores, info.num_subcores
    NW = NC * NS
    per_w = E // NW
    CHUNK = 80
    n_chunks = per_w // CHUNK

    mesh = plsc.VectorSubcoreMesh(core_axis_name="c", subcore_axis_name="s")

    @functools.partial(
        pl.kernel,
        out_type=jax.ShapeDtypeStruct((E, H), jnp.float32),
        mesh=mesh,
        scratch_types=[
            pltpu.VMEM((CHUNK,), jnp.int32),
            pltpu.VMEM((CHUNK, H), jnp.float32),
            pltpu.SemaphoreType.DMA,
        ],
    )
    def _gather(idx_hbm, g_hbm, out_hbm, idx_v, rows_v, sem):
        wid = lax.axis_index("s") * NC + lax.axis_index("c")
        base0 = wid * per_w

        def body(i, carry):
            base = base0 + i * CHUNK
            pltpu.sync_copy(idx_hbm.at[pl.ds(base, CHUNK)], idx_v)
            pltpu.async_copy(g_hbm.at[idx_v], rows_v, sem).wait()
            pltpu.sync_copy(rows_v, out_hbm.at[pl.ds(base, CHUNK)])
            return carry

        lax.fori_loop(0, n_chunks, body, 0)

    gathered = _gather(idxg.reshape(E), g)

    BLKN = 200
    grid3 = BN // BLKN
    out = pl.pallas_call(
        functools.partial(_stage3_body, BLKN, K, H),
        grid=(grid3,),
        in_specs=[
            pl.BlockSpec((BLKN * K, H), lambda i: (i, 0)),
            pl.BlockSpec((BLKN, H), lambda i: (i, 0)),
            pl.BlockSpec((H, OUT), lambda i: (0, 0)),
            pl.BlockSpec((1, OUT), lambda i: (0, 0)),
            pl.BlockSpec((1, OUT), lambda i: (0, 0)),
            pl.BlockSpec((1, OUT), lambda i: (0, 0)),
        ],
        out_specs=pl.BlockSpec((BLKN, OUT), lambda i: (i, 0)),
        out_shape=jax.ShapeDtypeStruct((BN, OUT), jnp.float32),
    )(gathered, a, W2, b2.reshape(1, OUT), gamma.reshape(1, OUT),
      beta.reshape(1, OUT))

    return out.reshape(B, N, OUT)


# speedup vs baseline: 27.0592x; 1.5301x over previous
import functools

import jax
import jax.numpy as jnp
from jax import lax
from jax.experimental import pallas as pl
from jax.experimental.pallas import tpu as pltpu
from jax.experimental.pallas import tpu_sc as plsc


_GELU_Q = (-7.834085e-10, 7.279727e-08, -2.9049525e-06, 6.5854394e-05,
           -9.506859e-04, 9.339054e-03, -6.5704457e-02, 3.9865878e-01)
_GELU_XC = 4.5


def _gelu(x):
    xc = jnp.clip(x, -_GELU_XC, _GELU_XC)
    s = xc * xc
    acc = jnp.full_like(s, _GELU_Q[0])
    for c in _GELU_Q[1:]:
        acc = acc * s + c
    return x * (acc * xc + 0.5)


def _stage1_body(n_per_batch, blk, h_ref, p_ref, i_ref, w1_ref, b1_ref,
                 a_ref, g_ref, ig_ref):
    C = h_ref.shape[1]
    hb = h_ref[...]
    w1a = w1_ref[0:C, :]
    w1b = w1_ref[C:2 * C, :]
    r = (p_ref[:, 0:1] * w1_ref[2 * C:2 * C + 1, :]
         + p_ref[:, 1:2] * w1_ref[2 * C + 1:2 * C + 2, :]
         + p_ref[:, 2:3] * w1_ref[2 * C + 2:2 * C + 3, :])
    g_ref[...] = jnp.dot(hb, w1b, preferred_element_type=jnp.float32) + r
    a_ref[...] = (jnp.dot(hb, w1a - w1b, preferred_element_type=jnp.float32)
                  + b1_ref[...] - r)
    pid = pl.program_id(0)
    ig_ref[...] = i_ref[...] + (pid * blk) // n_per_batch * n_per_batch


def _stage3_body(nodes, k, h_out, gath_ref, a_ref, w2_ref, b2_ref,
                 gm_ref, bt_ref, o_ref):
    z = gath_ref[...].reshape(nodes, k, h_out) + a_ref[...][:, None, :]
    x1 = _gelu(z).reshape(nodes * k, h_out)
    y = jnp.dot(x1, w2_ref[...], preferred_element_type=jnp.float32) + b2_ref[...]
    x2 = _gelu(y).reshape(nodes, k, h_out)
    m = jnp.max(x2, axis=1)
    mu = jnp.mean(m, axis=-1, keepdims=True)
    var = jnp.mean((m - mu) ** 2, axis=-1, keepdims=True)
    o_ref[...] = (m - mu) * lax.rsqrt(var + 1e-5) * gm_ref[...] + bt_ref[...]


def kernel(h, pos, idx, W1, b1, W2, b2, gamma, beta):
    B, N, C = h.shape
    K = idx.shape[-1]
    H = W1.shape[1]
    OUT = W2.shape[1]
    BN = B * N
    E = BN * K

    hf = h.reshape(BN, C)
    pf = pos.reshape(BN, 3)
    idxf = idx.reshape(BN, K)

    BLK1 = 1000
    grid1 = BN // BLK1
    a, g, idxg = pl.pallas_call(
        functools.partial(_stage1_body, N, BLK1),
        grid=(grid1,),
        in_specs=[
            pl.BlockSpec((BLK1, C), lambda i: (i, 0)),
            pl.BlockSpec((BLK1, 3), lambda i: (i, 0)),
            pl.BlockSpec((BLK1, K), lambda i: (i, 0)),
            pl.BlockSpec((2 * C + 3, H), lambda i: (0, 0)),
            pl.BlockSpec((1, H), lambda i: (0, 0)),
        ],
        out_specs=[
            pl.BlockSpec((BLK1, H), lambda i: (i, 0)),
            pl.BlockSpec((BLK1, H), lambda i: (i, 0)),
            pl.BlockSpec((BLK1, K), lambda i: (i, 0)),
        ],
        out_shape=[
            jax.ShapeDtypeStruct((BN, H), jnp.float32),
            jax.ShapeDtypeStruct((BN, H), jnp.float32),
            jax.ShapeDtypeStruct((BN, K), jnp.int32),
        ],
    )(hf, pf, idxf, W1, b1.reshape(1, H))

    info = plsc.get_sparse_core_info()
    NC, NS = info.num_cores, info.num_subcores
    NW = NC * NS
    per_w = E // NW
    CHUNK = 80
    NBUF = 5
    n_groups = per_w // (CHUNK * NBUF)

    mesh = plsc.VectorSubcoreMesh(core_axis_name="c", subcore_axis_name="s")

    @functools.partial(
        pl.kernel,
        out_type=jax.ShapeDtypeStruct((E, H), jnp.float32),
        mesh=mesh,
        scratch_types=[
            pltpu.VMEM((per_w,), jnp.int32),
            pltpu.VMEM((2, NBUF, CHUNK, H), jnp.float32),
            pltpu.SemaphoreType.DMA,
            pltpu.SemaphoreType.DMA,
        ],
    )
    def _gather(idx_hbm, g_hbm, out_hbm, idx_v, rows_v, sem_g, sem_s):
        wid = lax.axis_index("s") * NC + lax.axis_index("c")
        base0 = wid * per_w
        pltpu.sync_copy(idx_hbm.at[pl.ds(base0, per_w)], idx_v)

        def drain_stores(par):
            for b in range(NBUF):
                pltpu.make_async_copy(
                    rows_v.at[par, b],
                    out_hbm.at[pl.ds(base0, CHUNK)], sem_s).wait()

        def body(t, carry):
            par = lax.rem(t, 2)
            @pl.when(t >= 2)
            def _():
                drain_stores(par)
            gets = []
            for b in range(NBUF):
                off = (t * NBUF + b) * CHUNK
                gets.append(pltpu.async_copy(
                    g_hbm.at[idx_v.at[pl.ds(off, CHUNK)]],
                    rows_v.at[par, b], sem_g))
            for c in gets:
                c.wait()
            for b in range(NBUF):
                off = (t * NBUF + b) * CHUNK
                pltpu.async_copy(rows_v.at[par, b],
                                 out_hbm.at[pl.ds(base0 + off, CHUNK)], sem_s)
            return carry

        lax.fori_loop(0, n_groups, body, 0)
        drain_stores(lax.rem(jnp.int32(n_groups), 2))
        if n_groups >= 2:
            drain_stores(lax.rem(jnp.int32(n_groups) + 1, 2))

    gathered = _gather(idxg.reshape(E), g)

    BLKN = 200
    grid3 = BN // BLKN
    out = pl.pallas_call(
        functools.partial(_stage3_body, BLKN, K, H),
        grid=(grid3,),
        in_specs=[
            pl.BlockSpec((BLKN * K, H), lambda i: (i, 0)),
            pl.BlockSpec((BLKN, H), lambda i: (i, 0)),
            pl.BlockSpec((H, OUT), lambda i: (0, 0)),
            pl.BlockSpec((1, OUT), lambda i: (0, 0)),
            pl.BlockSpec((1, OUT), lambda i: (0, 0)),
            pl.BlockSpec((1, OUT), lambda i: (0, 0)),
        ],
        out_specs=pl.BlockSpec((BLKN, OUT), lambda i: (i, 0)),
        out_shape=jax.ShapeDtypeStruct((BN, OUT), jnp.float32),
    )(gathered, a, W2, b2.reshape(1, OUT), gamma.reshape(1, OUT),
      beta.reshape(1, OUT))

    return out.reshape(B, N, OUT)
